# baseline (speedup 1.0000x reference)
import functools

import jax
import jax.numpy as jnp
from jax import lax
from jax.experimental import pallas as pl
from jax.experimental.pallas import tpu as pltpu
from jax.experimental.pallas import tpu_sc as plsc

_N_ATOMS = 10000
_N_EDGES = 320000
_D = 128
_NS = 16
_STRIPE = 624
_STRIPE_LAST = _N_ATOMS - 15 * _STRIPE
_E_PER_TILE = _N_EDGES // _NS
_WIN = 96
_N_WIN = _E_PER_TILE // _WIN
_N_PAIR = _N_WIN // 2
_TAIL = _E_PER_TILE - _N_WIN * _WIN
_N_TRASH = 16
_ACC_ROWS = _N_ATOMS + _N_TRASH
_LANES = 16
_COLS = _D // _LANES


def _zero_buf(buf, n_rows):
    zero = jnp.zeros((_LANES,), jnp.float32)

    def body(r, carry):
        for k in range(_COLS):
            buf[r, pl.ds(k * _LANES, _LANES)] = zero
        return carry

    lax.fori_loop(0, n_rows, body, 0)


_mesh = plsc.VectorSubcoreMesh(core_axis_name="c", subcore_axis_name="s")


@functools.partial(
    pl.kernel,
    out_type=(
        jax.ShapeDtypeStruct((_N_ATOMS, _D), jnp.float32),
        jax.ShapeDtypeStruct((_N_ATOMS, _D), jnp.float32),
    ),
    mesh=_mesh,
    scratch_types=[
        pltpu.VMEM((_WIN, _D), jnp.float32),
        pltpu.VMEM((_WIN, _D), jnp.float32),
        pltpu.VMEM((_WIN, _D), jnp.float32),
        pltpu.VMEM((_WIN, _D), jnp.float32),
        pltpu.VMEM((_WIN,), jnp.int32),
        pltpu.VMEM((_WIN,), jnp.int32),
        pltpu.VMEM((_WIN,), jnp.int32),
        pltpu.VMEM((_WIN,), jnp.int32),
        pltpu.SemaphoreType.DMA,
        pltpu.SemaphoreType.DMA,
        pltpu.SemaphoreType.DMA,
        pltpu.SemaphoreType.DMA,
        pltpu.VMEM_SHARED((_ACC_ROWS, _D), jnp.float32),
    ],
)
def _seg_accum(
    m1, m2, id1, id2, id3, id4, a_out, b_out,
    vals0, vals1, nvals0, nvals1, ida0, ida1, idb0, idb1,
    insem0, insem1, scsem0, scsem1, acc,
):
    c = lax.axis_index("c")
    s = lax.axis_index("s")
    row0 = s * _STRIPE
    is_last = s == _NS - 1

    def run_core(mat, ia, ib):
        slots = (
            (vals0, nvals0, ida0, idb0, insem0, scsem0),
            (vals1, nvals1, ida1, idb1, insem1, scsem1),
        )

        def start_in(slot, w):
            vals, _, ida, idb, insem, _ = slots[slot]
            base = s * _E_PER_TILE + w * _WIN
            pltpu.async_copy(mat.at[pl.ds(base, _WIN)], vals, insem)
            pltpu.async_copy(ia.at[pl.ds(base, _WIN)], ida, insem)
            pltpu.async_copy(ib.at[pl.ds(base, _WIN)], idb, insem)

        def drain_in(slot, w):
            vals, _, ida, idb, insem, _ = slots[slot]
            base = s * _E_PER_TILE + w * _WIN
            pltpu.make_async_copy(mat.at[pl.ds(base, _WIN)], vals, insem).wait()
            pltpu.make_async_copy(ia.at[pl.ds(base, _WIN)], ida, insem).wait()
            pltpu.make_async_copy(ib.at[pl.ds(base, _WIN)], idb, insem).wait()

        def start_scat(slot):
            vals, nvals, ida, idb, _, scsem = slots[slot]
            pltpu.async_copy(vals, acc.at[ida], scsem, add=True)

            def neg(r, carry):
                for k in range(_COLS):
                    sl = pl.ds(k * _LANES, _LANES)
                    nvals[r, sl] = -vals[r, sl]
                return carry

            lax.fori_loop(0, _WIN, neg, 0, unroll=4)
            pltpu.async_copy(nvals, acc.at[idb], scsem, add=True)

        def drain_scat(slot):
            vals, nvals, ida, idb, _, scsem = slots[slot]
            pltpu.make_async_copy(vals, acc.at[ida], scsem).wait()
            pltpu.make_async_copy(nvals, acc.at[idb], scsem).wait()

        start_in(0, 0)

        _zero_buf(nvals0, _WIN)
        n_zfull = _STRIPE // _WIN

        def zbody(j, carry):
            pltpu.sync_copy(nvals0, acc.at[pl.ds(row0 + j * _WIN, _WIN)])
            return carry

        lax.fori_loop(0, n_zfull, zbody, 0)
        rem = _STRIPE - n_zfull * _WIN
        rem_last = _STRIPE_LAST + _N_TRASH - n_zfull * _WIN

        @pl.when(jnp.logical_not(is_last))
        def _():
            pltpu.sync_copy(
                nvals0.at[pl.ds(0, rem)],
                acc.at[pl.ds(row0 + n_zfull * _WIN, rem)],
            )

        @pl.when(is_last)
        def _():
            pltpu.sync_copy(
                nvals0.at[pl.ds(0, rem_last)],
                acc.at[pl.ds(row0 + n_zfull * _WIN, rem_last)],
            )

        plsc.subcore_barrier()

        def body(p, carry):
            w = 2 * p

            @pl.when(p > 0)
            def _():
                drain_scat(1)

            start_in(1, w + 1)
            drain_in(0, w)
            start_scat(0)

            @pl.when(p < _N_PAIR - 1)
            def _():
                drain_scat(0)
                start_in(0, w + 2)

            drain_in(1, w + 1)
            start_scat(1)
            return carry

        lax.fori_loop(0, _N_PAIR, body, 0)

        drain_scat(0)
        tbase = s * _E_PER_TILE + _N_WIN * _WIN
        pltpu.sync_copy(mat.at[pl.ds(tbase, _TAIL)], vals0.at[pl.ds(0, _TAIL)])
        pltpu.sync_copy(ia.at[pl.ds(tbase, _TAIL)], ida0.at[pl.ds(0, _TAIL)])
        pltpu.sync_copy(ib.at[pl.ds(tbase, _TAIL)], idb0.at[pl.ds(0, _TAIL)])
        trash = lax.iota(jnp.int32, _LANES) + _N_ATOMS
        for off in range(_TAIL, _WIN, _LANES):
            ida0[pl.ds(off, _LANES)] = trash
            idb0[pl.ds(off, _LANES)] = trash
        pltpu.async_copy(vals0, acc.at[ida0], scsem0, add=True)

        def tneg(r, carry):
            for k in range(_COLS):
                sl = pl.ds(k * _LANES, _LANES)
                nvals0[r, sl] = -vals0[r, sl]
            return carry

        lax.fori_loop(0, _TAIL, tneg, 0, unroll=4)
        pltpu.async_copy(nvals0, acc.at[idb0], scsem0, add=True)

        drain_scat(1)
        drain_scat(0)

    @pl.when(c == 0)
    def _():
        run_core(m1, id1, id3)

    @pl.when(c == 1)
    def _():
        run_core(m2, id2, id4)

    plsc.subcore_barrier()

    for core, out in ((0, a_out), (1, b_out)):

        @pl.when(jnp.logical_and(c == core, jnp.logical_not(is_last)))
        def _(out=out):
            pltpu.sync_copy(
                acc.at[pl.ds(row0, _STRIPE)], out.at[pl.ds(row0, _STRIPE)]
            )

        @pl.when(jnp.logical_and(c == core, is_last))
        def _(out=out):
            pltpu.sync_copy(
                acc.at[pl.ds(row0, _STRIPE_LAST)],
                out.at[pl.ds(row0, _STRIPE_LAST)],
            )


_BLK = 2000


def _hdot_body(h_ref, w3_ref, g_ref):
    g_ref[...] = jnp.dot(
        h_ref[...], w3_ref[...], preferred_element_type=jnp.float32
    )


def _hdot(h, w3):
    n_blk = _N_ATOMS // _BLK
    return pl.pallas_call(
        _hdot_body,
        grid=(n_blk,),
        in_specs=[
            pl.BlockSpec((_BLK, _D), lambda i: (i, 0)),
            pl.BlockSpec((_D, _D), lambda i: (0, 0)),
        ],
        out_specs=pl.BlockSpec((_BLK, _D), lambda i: (i, 0)),
        out_shape=jax.ShapeDtypeStruct((_N_ATOMS, _D), jnp.float32),
    )(h, w3)


def _mlp_body(a_ref, b_ref, g_ref, w12_ref, o_ref):
    w12 = w12_ref[...]
    o_ref[...] = (
        g_ref[...]
        + jnp.dot(a_ref[...], w12[0:_D, :], preferred_element_type=jnp.float32)
        + jnp.dot(b_ref[...], w12[_D:, :], preferred_element_type=jnp.float32)
    )


def _mlp(a, b, g, W12):
    n_blk = _N_ATOMS // _BLK
    return pl.pallas_call(
        _mlp_body,
        grid=(n_blk,),
        in_specs=[
            pl.BlockSpec((_BLK, _D), lambda i: (i, 0)),
            pl.BlockSpec((_BLK, _D), lambda i: (i, 0)),
            pl.BlockSpec((_BLK, _D), lambda i: (i, 0)),
            pl.BlockSpec((2 * _D, _D), lambda i: (0, 0)),
        ],
        out_specs=pl.BlockSpec((_BLK, _D), lambda i: (i, 0)),
        out_shape=jax.ShapeDtypeStruct((_N_ATOMS, _D), jnp.float32),
    )(a, b, g, W12)


def kernel(h, m1, m2, id1, id2, id3, id4, W):
    g = _hdot(h, W[2 * _D :])
    a, b = _seg_accum(
        m1,
        m2,
        id1.astype(jnp.int32),
        id2.astype(jnp.int32),
        id3.astype(jnp.int32),
        id4.astype(jnp.int32),
    )
    return _mlp(a, b, g, W[0 : 2 * _D])


# speedup vs baseline: 2.5181x; 2.5181x over previous
import functools

import jax
import jax.numpy as jnp
from jax import lax
from jax.experimental import pallas as pl
from jax.experimental.pallas import tpu as pltpu
from jax.experimental.pallas import tpu_sc as plsc

_N_ATOMS = 10000
_N_EDGES = 320000
_D = 128
_NS = 16
_STRIPE = 624
_STRIPE_LAST = _N_ATOMS - 15 * _STRIPE
_E_PER_TILE = _N_EDGES // _NS
_WIN = 64
_N_WIN = 312
_N_TRIPLE = _N_WIN // 3
_TAIL = _E_PER_TILE - _N_WIN * _WIN
_N_TRASH = 16
_ACC_ROWS = _N_ATOMS + _N_TRASH
_LANES = 16
_COLS = _D // _LANES


def _zero_buf(buf, n_rows):
    zero = jnp.zeros((_LANES,), jnp.float32)

    def body(r, carry):
        for k in range(_COLS):
            buf[r, pl.ds(k * _LANES, _LANES)] = zero
        return carry

    lax.fori_loop(0, n_rows, body, 0)


_mesh = plsc.VectorSubcoreMesh(core_axis_name="c", subcore_axis_name="s")


@functools.partial(
    pl.kernel,
    out_type=(
        jax.ShapeDtypeStruct((_N_ATOMS, _D), jnp.float32),
        jax.ShapeDtypeStruct((_N_ATOMS, _D), jnp.float32),
    ),
    mesh=_mesh,
    scratch_types=[
        pltpu.VMEM((_WIN, _D), jnp.float32),
        pltpu.VMEM((_WIN, _D), jnp.float32),
        pltpu.VMEM((_WIN, _D), jnp.float32),
        pltpu.VMEM((_WIN, _D), jnp.float32),
        pltpu.VMEM((_WIN, _D), jnp.float32),
        pltpu.VMEM((_WIN, _D), jnp.float32),
        pltpu.VMEM((_WIN,), jnp.int32),
        pltpu.VMEM((_WIN,), jnp.int32),
        pltpu.VMEM((_WIN,), jnp.int32),
        pltpu.VMEM((_WIN,), jnp.int32),
        pltpu.VMEM((_WIN,), jnp.int32),
        pltpu.VMEM((_WIN,), jnp.int32),
        pltpu.SemaphoreType.DMA,
        pltpu.SemaphoreType.DMA,
        pltpu.SemaphoreType.DMA,
        pltpu.SemaphoreType.DMA,
        pltpu.SemaphoreType.DMA,
        pltpu.SemaphoreType.DMA,
        pltpu.VMEM_SHARED((_ACC_ROWS, _D), jnp.float32),
    ],
)
def _seg_accum(
    m1, m2, id1, id2, id3, id4, a_out, b_out,
    vals0, vals1, vals2, nvals0, nvals1, nvals2,
    ida0, ida1, ida2, idb0, idb1, idb2,
    insem0, insem1, insem2, scsem0, scsem1, scsem2, acc,
):
    c = lax.axis_index("c")
    s = lax.axis_index("s")
    row0 = s * _STRIPE
    is_last = s == _NS - 1

    def run_core(mat, ia, ib):
        slots = (
            (vals0, nvals0, ida0, idb0, insem0, scsem0),
            (vals1, nvals1, ida1, idb1, insem1, scsem1),
            (vals2, nvals2, ida2, idb2, insem2, scsem2),
        )

        def start_in(slot, w):
            vals, _, ida, idb, insem, _ = slots[slot]
            base = s * _E_PER_TILE + w * _WIN
            pltpu.async_copy(mat.at[pl.ds(base, _WIN)], vals, insem)
            pltpu.async_copy(ia.at[pl.ds(base, _WIN)], ida, insem)
            pltpu.async_copy(ib.at[pl.ds(base, _WIN)], idb, insem)

        def drain_in(slot, w):
            vals, _, ida, idb, insem, _ = slots[slot]
            base = s * _E_PER_TILE + w * _WIN
            pltpu.make_async_copy(mat.at[pl.ds(base, _WIN)], vals, insem).wait()
            pltpu.make_async_copy(ia.at[pl.ds(base, _WIN)], ida, insem).wait()
            pltpu.make_async_copy(ib.at[pl.ds(base, _WIN)], idb, insem).wait()

        def start_scat(slot):
            vals, nvals, ida, idb, _, scsem = slots[slot]
            pltpu.async_copy(vals, acc.at[ida], scsem, add=True)

            def neg(r, carry):
                for k in range(_COLS):
                    sl = pl.ds(k * _LANES, _LANES)
                    nvals[r, sl] = -vals[r, sl]
                return carry

            lax.fori_loop(0, _WIN, neg, 0)
            pltpu.async_copy(nvals, acc.at[idb], scsem, add=True)

        def drain_scat(slot):
            vals, nvals, ida, idb, _, scsem = slots[slot]
            pltpu.make_async_copy(vals, acc.at[ida], scsem).wait()
            pltpu.make_async_copy(nvals, acc.at[idb], scsem).wait()

        start_in(0, 0)

        _zero_buf(nvals0, _WIN)
        n_zfull = jnp.where(is_last, 10, 9)

        def zbody(j, carry):
            pltpu.sync_copy(nvals0, acc.at[pl.ds(row0 + j * _WIN, _WIN)])
            return carry

        lax.fori_loop(0, n_zfull, zbody, 0)
        rem = _STRIPE - 9 * _WIN
        rem_last = _STRIPE_LAST + _N_TRASH - 10 * _WIN

        @pl.when(jnp.logical_not(is_last))
        def _():
            pltpu.sync_copy(
                nvals0.at[pl.ds(0, rem)],
                acc.at[pl.ds(row0 + 9 * _WIN, rem)],
            )

        @pl.when(is_last)
        def _():
            pltpu.sync_copy(
                nvals0.at[pl.ds(0, rem_last)],
                acc.at[pl.ds(row0 + 10 * _WIN, rem_last)],
            )

        plsc.subcore_barrier()

        def body(p, carry):
            w = 3 * p

            @pl.when(p > 0)
            def _():
                drain_scat(1)

            start_in(1, w + 1)
            drain_in(0, w)
            start_scat(0)

            @pl.when(p > 0)
            def _():
                drain_scat(2)

            start_in(2, w + 2)
            drain_in(1, w + 1)
            start_scat(1)

            @pl.when(p < _N_TRIPLE - 1)
            def _():
                drain_scat(0)
                start_in(0, w + 3)

            drain_in(2, w + 2)
            start_scat(2)
            return carry

        lax.fori_loop(0, _N_TRIPLE, body, 0)

        drain_scat(0)
        tbase = s * _E_PER_TILE + _N_WIN * _WIN
        pltpu.sync_copy(mat.at[pl.ds(tbase, _TAIL)], vals0.at[pl.ds(0, _TAIL)])
        pltpu.sync_copy(ia.at[pl.ds(tbase, _TAIL)], ida0.at[pl.ds(0, _TAIL)])
        pltpu.sync_copy(ib.at[pl.ds(tbase, _TAIL)], idb0.at[pl.ds(0, _TAIL)])
        trash = lax.iota(jnp.int32, _LANES) + _N_ATOMS
        for off in range(_TAIL, _WIN, _LANES):
            ida0[pl.ds(off, _LANES)] = trash
            idb0[pl.ds(off, _LANES)] = trash
        pltpu.async_copy(vals0, acc.at[ida0], scsem0, add=True)

        def tneg(r, carry):
            for k in range(_COLS):
                sl = pl.ds(k * _LANES, _LANES)
                nvals0[r, sl] = -vals0[r, sl]
            return carry

        lax.fori_loop(0, _TAIL, tneg, 0)
        pltpu.async_copy(nvals0, acc.at[idb0], scsem0, add=True)

        drain_scat(1)
        drain_scat(2)
        drain_scat(0)

    @pl.when(c == 0)
    def _():
        run_core(m1, id1, id3)

    @pl.when(c == 1)
    def _():
        run_core(m2, id2, id4)

    plsc.subcore_barrier()

    for core, out in ((0, a_out), (1, b_out)):

        @pl.when(jnp.logical_and(c == core, jnp.logical_not(is_last)))
        def _(out=out):
            pltpu.sync_copy(
                acc.at[pl.ds(row0, _STRIPE)], out.at[pl.ds(row0, _STRIPE)]
            )

        @pl.when(jnp.logical_and(c == core, is_last))
        def _(out=out):
            pltpu.sync_copy(
                acc.at[pl.ds(row0, _STRIPE_LAST)],
                out.at[pl.ds(row0, _STRIPE_LAST)],
            )


_BLK = 2000


def _hdot_body(h_ref, w3_ref, g_ref):
    g_ref[...] = jnp.dot(
        h_ref[...], w3_ref[...], preferred_element_type=jnp.float32
    )


def _hdot(h, w3):
    n_blk = _N_ATOMS // _BLK
    return pl.pallas_call(
        _hdot_body,
        grid=(n_blk,),
        in_specs=[
            pl.BlockSpec((_BLK, _D), lambda i: (i, 0)),
            pl.BlockSpec((_D, _D), lambda i: (0, 0)),
        ],
        out_specs=pl.BlockSpec((_BLK, _D), lambda i: (i, 0)),
        out_shape=jax.ShapeDtypeStruct((_N_ATOMS, _D), jnp.float32),
    )(h, w3)


def _mlp_body(a_ref, b_ref, g_ref, w12_ref, o_ref):
    w12 = w12_ref[...]
    o_ref[...] = (
        g_ref[...]
        + jnp.dot(a_ref[...], w12[0:_D, :], preferred_element_type=jnp.float32)
        + jnp.dot(b_ref[...], w12[_D:, :], preferred_element_type=jnp.float32)
    )


def _mlp(a, b, g, W12):
    n_blk = _N_ATOMS // _BLK
    return pl.pallas_call(
        _mlp_body,
        grid=(n_blk,),
        in_specs=[
            pl.BlockSpec((_BLK, _D), lambda i: (i, 0)),
            pl.BlockSpec((_BLK, _D), lambda i: (i, 0)),
            pl.BlockSpec((_BLK, _D), lambda i: (i, 0)),
            pl.BlockSpec((2 * _D, _D), lambda i: (0, 0)),
        ],
        out_specs=pl.BlockSpec((_BLK, _D), lambda i: (i, 0)),
        out_shape=jax.ShapeDtypeStruct((_N_ATOMS, _D), jnp.float32),
    )(a, b, g, W12)


def kernel(h, m1, m2, id1, id2, id3, id4, W):
    g = _hdot(h, W[2 * _D :])
    a, b = _seg_accum(
        m1,
        m2,
        id1.astype(jnp.int32),
        id2.astype(jnp.int32),
        id3.astype(jnp.int32),
        id4.astype(jnp.int32),
    )
    return _mlp(a, b, g, W[0 : 2 * _D])


# speedup vs baseline: 2.6033x; 1.0338x over previous
import functools

import jax
import jax.numpy as jnp
from jax import lax
from jax.experimental import pallas as pl
from jax.experimental.pallas import tpu as pltpu
from jax.experimental.pallas import tpu_sc as plsc

_N_ATOMS = 10000
_N_EDGES = 320000
_D = 128
_NS = 16
_STRIPE = 624
_STRIPE_LAST = _N_ATOMS - 15 * _STRIPE
_E_PER_TILE = _N_EDGES // _NS
_WIN = 96
_N_WIN = _E_PER_TILE // _WIN
_N_PAIR = _N_WIN // 2
_TAIL = _E_PER_TILE - _N_WIN * _WIN
_N_TRASH = 16
_ACC_ROWS = _N_ATOMS + _N_TRASH
_LANES = 16
_COLS = _D // _LANES


def _zero_buf(buf, n_rows):
    zero = jnp.zeros((_LANES,), jnp.float32)

    def body(r, carry):
        for k in range(_COLS):
            buf[r, pl.ds(k * _LANES, _LANES)] = zero
        return carry

    lax.fori_loop(0, n_rows, body, 0)


_mesh = plsc.VectorSubcoreMesh(core_axis_name="c", subcore_axis_name="s")


@functools.partial(
    pl.kernel,
    out_type=(
        jax.ShapeDtypeStruct((_N_ATOMS, _D), jnp.float32),
        jax.ShapeDtypeStruct((_N_ATOMS, _D), jnp.float32),
    ),
    mesh=_mesh,
    scratch_types=[
        pltpu.VMEM((_WIN, _D), jnp.float32),
        pltpu.VMEM((_WIN, _D), jnp.float32),
        pltpu.VMEM((_WIN, _D), jnp.float32),
        pltpu.VMEM((_WIN, _D), jnp.float32),
        pltpu.VMEM((_WIN,), jnp.int32),
        pltpu.VMEM((_WIN,), jnp.int32),
        pltpu.VMEM((_WIN,), jnp.int32),
        pltpu.VMEM((_WIN,), jnp.int32),
        pltpu.SemaphoreType.DMA,
        pltpu.SemaphoreType.DMA,
        pltpu.SemaphoreType.DMA,
        pltpu.SemaphoreType.DMA,
        pltpu.VMEM_SHARED((_ACC_ROWS, _D), jnp.float32),
    ],
)
def _seg_accum(
    m1, m2, id1, id2, id3, id4, a_out, b_out,
    vals0, vals1, nvals0, nvals1, ida0, ida1, idb0, idb1,
    insem0, insem1, scsem0, scsem1, acc,
):
    c = lax.axis_index("c")
    s = lax.axis_index("s")
    row0 = s * _STRIPE
    is_last = s == _NS - 1

    def run_core(mat, ia, ib):
        slots = (
            (vals0, nvals0, ida0, idb0, insem0, scsem0),
            (vals1, nvals1, ida1, idb1, insem1, scsem1),
        )

        def start_in(slot, w):
            vals, _, ida, idb, insem, _ = slots[slot]
            base = s * _E_PER_TILE + w * _WIN
            pltpu.async_copy(mat.at[pl.ds(base, _WIN)], vals, insem)
            pltpu.async_copy(ia.at[pl.ds(base, _WIN)], ida, insem)
            pltpu.async_copy(ib.at[pl.ds(base, _WIN)], idb, insem)

        def drain_in(slot, w):
            vals, _, ida, idb, insem, _ = slots[slot]
            base = s * _E_PER_TILE + w * _WIN
            pltpu.make_async_copy(mat.at[pl.ds(base, _WIN)], vals, insem).wait()
            pltpu.make_async_copy(ia.at[pl.ds(base, _WIN)], ida, insem).wait()
            pltpu.make_async_copy(ib.at[pl.ds(base, _WIN)], idb, insem).wait()

        def start_scat(slot):
            vals, nvals, ida, idb, _, scsem = slots[slot]
            pltpu.async_copy(vals, acc.at[ida], scsem, priority=1, add=True)

            def neg(r, carry):
                for k in range(_COLS):
                    sl = pl.ds(k * _LANES, _LANES)
                    nvals[r, sl] = -vals[r, sl]
                return carry

            lax.fori_loop(0, _WIN, neg, 0)
            pltpu.async_copy(nvals, acc.at[idb], scsem, priority=1, add=True)

        def drain_scat(slot):
            vals, nvals, ida, idb, _, scsem = slots[slot]
            pltpu.make_async_copy(vals, acc.at[ida], scsem).wait()
            pltpu.make_async_copy(nvals, acc.at[idb], scsem).wait()

        start_in(0, 0)

        _zero_buf(nvals0, _WIN)
        n_zfull = _STRIPE // _WIN

        def zbody(j, carry):
            pltpu.sync_copy(nvals0, acc.at[pl.ds(row0 + j * _WIN, _WIN)])
            return carry

        lax.fori_loop(0, n_zfull, zbody, 0)
        rem = _STRIPE - n_zfull * _WIN
        rem_last = _STRIPE_LAST + _N_TRASH - n_zfull * _WIN

        @pl.when(jnp.logical_not(is_last))
        def _():
            pltpu.sync_copy(
                nvals0.at[pl.ds(0, rem)],
                acc.at[pl.ds(row0 + n_zfull * _WIN, rem)],
            )

        @pl.when(is_last)
        def _():
            pltpu.sync_copy(
                nvals0.at[pl.ds(0, rem_last)],
                acc.at[pl.ds(row0 + n_zfull * _WIN, rem_last)],
            )

        plsc.subcore_barrier()

        def body(p, carry):
            w = 2 * p

            @pl.when(p > 0)
            def _():
                drain_scat(1)

            start_in(1, w + 1)
            drain_in(0, w)
            start_scat(0)

            @pl.when(p < _N_PAIR - 1)
            def _():
                drain_scat(0)
                start_in(0, w + 2)

            drain_in(1, w + 1)
            start_scat(1)
            return carry

        lax.fori_loop(0, _N_PAIR, body, 0)

        drain_scat(0)
        tbase = s * _E_PER_TILE + _N_WIN * _WIN
        pltpu.sync_copy(mat.at[pl.ds(tbase, _TAIL)], vals0.at[pl.ds(0, _TAIL)])
        pltpu.sync_copy(ia.at[pl.ds(tbase, _TAIL)], ida0.at[pl.ds(0, _TAIL)])
        pltpu.sync_copy(ib.at[pl.ds(tbase, _TAIL)], idb0.at[pl.ds(0, _TAIL)])
        trash = lax.iota(jnp.int32, _LANES) + _N_ATOMS
        for off in range(_TAIL, _WIN, _LANES):
            ida0[pl.ds(off, _LANES)] = trash
            idb0[pl.ds(off, _LANES)] = trash
        pltpu.async_copy(vals0, acc.at[ida0], scsem0, add=True)

        def tneg(r, carry):
            for k in range(_COLS):
                sl = pl.ds(k * _LANES, _LANES)
                nvals0[r, sl] = -vals0[r, sl]
            return carry

        lax.fori_loop(0, _TAIL, tneg, 0)
        pltpu.async_copy(nvals0, acc.at[idb0], scsem0, add=True)

        drain_scat(1)
        drain_scat(0)

    @pl.when(c == 0)
    def _():
        run_core(m1, id1, id3)

    @pl.when(c == 1)
    def _():
        run_core(m2, id2, id4)

    plsc.subcore_barrier()

    for core, out in ((0, a_out), (1, b_out)):

        @pl.when(jnp.logical_and(c == core, jnp.logical_not(is_last)))
        def _(out=out):
            pltpu.sync_copy(
                acc.at[pl.ds(row0, _STRIPE)], out.at[pl.ds(row0, _STRIPE)]
            )

        @pl.when(jnp.logical_and(c == core, is_last))
        def _(out=out):
            pltpu.sync_copy(
                acc.at[pl.ds(row0, _STRIPE_LAST)],
                out.at[pl.ds(row0, _STRIPE_LAST)],
            )


_BLK = 2000


def _hdot_body(h_ref, w3_ref, g_ref):
    g_ref[...] = jnp.dot(
        h_ref[...], w3_ref[...], preferred_element_type=jnp.float32
    )


def _hdot(h, w3):
    n_blk = _N_ATOMS // _BLK
    return pl.pallas_call(
        _hdot_body,
        grid=(n_blk,),
        in_specs=[
            pl.BlockSpec((_BLK, _D), lambda i: (i, 0)),
            pl.BlockSpec((_D, _D), lambda i: (0, 0)),
        ],
        out_specs=pl.BlockSpec((_BLK, _D), lambda i: (i, 0)),
        out_shape=jax.ShapeDtypeStruct((_N_ATOMS, _D), jnp.float32),
    )(h, w3)


def _mlp_body(a_ref, b_ref, g_ref, w12_ref, o_ref):
    w12 = w12_ref[...]
    o_ref[...] = (
        g_ref[...]
        + jnp.dot(a_ref[...], w12[0:_D, :], preferred_element_type=jnp.float32)
        + jnp.dot(b_ref[...], w12[_D:, :], preferred_element_type=jnp.float32)
    )


def _mlp(a, b, g, W12):
    n_blk = _N_ATOMS // _BLK
    return pl.pallas_call(
        _mlp_body,
        grid=(n_blk,),
        in_specs=[
            pl.BlockSpec((_BLK, _D), lambda i: (i, 0)),
            pl.BlockSpec((_BLK, _D), lambda i: (i, 0)),
            pl.BlockSpec((_BLK, _D), lambda i: (i, 0)),
            pl.BlockSpec((2 * _D, _D), lambda i: (0, 0)),
        ],
        out_specs=pl.BlockSpec((_BLK, _D), lambda i: (i, 0)),
        out_shape=jax.ShapeDtypeStruct((_N_ATOMS, _D), jnp.float32),
    )(a, b, g, W12)


def kernel(h, m1, m2, id1, id2, id3, id4, W):
    g = _hdot(h, W[2 * _D :])
    a, b = _seg_accum(
        m1,
        m2,
        id1.astype(jnp.int32),
        id2.astype(jnp.int32),
        id3.astype(jnp.int32),
        id4.astype(jnp.int32),
    )
    return _mlp(a, b, g, W[0 : 2 * _D])
